# Initial kernel scaffold; baseline (speedup 1.0000x reference)
import jax
import jax.numpy as jnp
from jax.experimental import pallas as pl


def kernel(x, adj, W_mu, b_mu, W_sig, b_sig):
    raise NotImplementedError("write your pallas kernel here")


# speedup vs baseline: 1.5010x; 1.5010x over previous
import jax
import jax.numpy as jnp
from jax.experimental import pallas as pl
from jax.experimental.pallas import tpu as pltpu

_BM = 400


def _support_kernel(x_ref, w_ref, out_ref):
    out_ref[...] = jnp.dot(
        x_ref[...], w_ref[...], preferred_element_type=jnp.float32
    ).astype(jnp.bfloat16)


def _main_kernel(adj_ref, sup_ref, eps_ref, b_ref, out_ref):
    a = adj_ref[...].astype(jnp.bfloat16)
    acc = jnp.dot(a, sup_ref[...], preferred_element_type=jnp.float32)
    acc = acc + b_ref[...]
    f = out_ref.shape[1]
    mu = acc[:, :f]
    log_sig = acc[:, f:]
    out_ref[...] = mu + eps_ref[...] * jnp.exp(log_sig)


def _forward(x, adj, W_mu, b_mu, W_sig, b_sig, interpret=False):
    n, fin = x.shape
    fout = W_mu.shape[1]
    wcat = jnp.concatenate([W_mu, W_sig], axis=1)
    bcat = jnp.concatenate([b_mu, b_sig])[None, :]

    bm = min(_BM, n)
    nb = n // bm

    support = pl.pallas_call(
        _support_kernel,
        grid=(nb,),
        in_specs=[
            pl.BlockSpec((bm, fin), lambda i: (i, 0)),
            pl.BlockSpec((fin, 2 * fout), lambda i: (0, 0)),
        ],
        out_specs=pl.BlockSpec((bm, 2 * fout), lambda i: (i, 0)),
        out_shape=jax.ShapeDtypeStruct((n, 2 * fout), jnp.bfloat16),
        interpret=interpret,
    )(x, wcat)

    eps = jax.random.normal(jax.random.key(42), (n, fout), dtype=jnp.float32)

    z = pl.pallas_call(
        _main_kernel,
        grid=(nb,),
        in_specs=[
            pl.BlockSpec((bm, n), lambda i: (i, 0)),
            pl.BlockSpec((n, 2 * fout), lambda i: (0, 0)),
            pl.BlockSpec((bm, fout), lambda i: (i, 0)),
            pl.BlockSpec((1, 2 * fout), lambda i: (0, 0)),
        ],
        out_specs=pl.BlockSpec((bm, fout), lambda i: (i, 0)),
        out_shape=jax.ShapeDtypeStruct((n, fout), jnp.float32),
        compiler_params=pltpu.CompilerParams(
            dimension_semantics=("arbitrary",)
        ),
        interpret=interpret,
    )(adj, support, eps, bcat)
    return z


def kernel(x, adj, W_mu, b_mu, W_sig, b_sig):
    return _forward(x, adj, W_mu, b_mu, W_sig, b_sig)
